# Initial kernel scaffold; baseline (speedup 1.0000x reference)
import jax
import jax.numpy as jnp
from jax.experimental import pallas as pl


def kernel(px, lx, pp_mat, pl_mat, ll_mat, res_idx, prot_g_idx, lig_g_idx, Win_p, bin_p, gin_p, bein_p, Win_l, bin_l, gin_l, bein_l, Wpg, bpg, gpg, bepg, Wlg, blg, glg, belg, e1_W, e1_b, e1_g, e1_be, e1_wp, e1_bp, e1_wl, e1_bl, e2_W, e2_b, e2_g, e2_be, e2_wp, e2_bp, e2_wl, e2_bl, Wemb, bemb, gemb, beemb, Wout, bout):
    raise NotImplementedError("write your pallas kernel here")


# speedup vs baseline: 1.2833x; 1.2833x over previous
import functools

import jax
import jax.numpy as jnp
from jax.experimental import pallas as pl
from jax.experimental.pallas import tpu as pltpu

_EPS = 1e-5
F32 = jnp.float32


def _q(x):
    i = jax.lax.bitcast_convert_type(x, jnp.int32)
    lsb = jax.lax.shift_right_logical(i, 16) & 1
    rounded = i + 0x7FFF + lsb
    masked = rounded & jnp.int32(-65536)
    return jax.lax.bitcast_convert_type(masked, F32)


def _bn_relu(h, g, be):
    mu = jnp.mean(h, axis=0, keepdims=True)
    var = jnp.mean((h - mu) ** 2, axis=0, keepdims=True)
    return jnp.maximum(g * (h - mu) / jnp.sqrt(var + _EPS) + be, 0.0)


def _dot(a, b):
    return jnp.dot(a, b, preferred_element_type=F32)


def _k_in_p(px_ref, w_ref, g_ref, be_ref, o_ref):
    h = _dot(px_ref[...], w_ref[...])
    o_ref[...] = _bn_relu(h, g_ref[...], be_ref[...])


def _k_pp(pp_ref, px1_ref, o_ref):
    o_ref[...] = _q(_dot(pp_ref[...], px1_ref[...]))


def _k_gather(t_ref, wpg_ref, idx_ref, g_ref, be_ref, o_ref):
    u = _dot(t_ref[...], wpg_ref[...])
    mu = jnp.mean(u, axis=0, keepdims=True)
    var = jnp.mean((u - mu) ** 2, axis=0, keepdims=True)
    px2 = _q(jnp.maximum(
        g_ref[...] * (u - mu) / jnp.sqrt(var + _EPS) + be_ref[...], 0.0))
    idx = idx_ref[...]
    acc = jnp.zeros((1600, 256), F32)
    for k in range(10):
        iota = jax.lax.broadcasted_iota(jnp.int32, (1600, 480), 1) + k * 480
        oh = (idx == iota).astype(F32)
        acc = acc + _dot(oh, px2[k * 480:(k + 1) * 480, :])
    o_ref[...] = acc


def _k_lig(lx_ref, winl_ref, ginl_ref, beinl_ref, ll_ref, wlg_ref, glg_ref,
           belg_ref, o_ref):
    h = _bn_relu(_dot(lx_ref[...], winl_ref[...]), ginl_ref[...], beinl_ref[...])
    tl = _q(_dot(ll_ref[...], h))
    o_ref[...] = _q(_bn_relu(_dot(tl, wlg_ref[...]), glg_ref[...], belg_ref[...]))


def _k_eirl(pl_ref, u_ref, x_ref, w_ref, g_ref, be_ref, wp_ref, bp_ref,
            wl_ref, bl_ref, ixg_ref, lxg_ref, a_ref, s2_ref, *, c):
    t = pl.program_id(0)
    p1 = pl_ref[...]

    @pl.when(t == 0)
    def _():
        s2_ref[...] = p1 * p1
        a_ref[...] = p1
        ixg_ref[...] = jnp.zeros_like(ixg_ref)
        lxg_ref[...] = jnp.zeros_like(lxg_ref)

    @pl.when(t == 1)
    def _():
        a_ref[...] = p1 * s2_ref[...]

    @pl.when(t == 2)
    def _():
        x4 = s2_ref[...] * s2_ref[...]
        a_ref[...] = s2_ref[...] * x4

    @pl.when(t == 3)
    def _():
        x4 = s2_ref[...] * s2_ref[...]
        a_ref[...] = p1 * (x4 * x4)

    @pl.when(t == 4)
    def _():
        x4 = s2_ref[...] * s2_ref[...]
        a_ref[...] = x4 * (x4 * x4)

    a = a_ref[...]
    u = _q(u_ref[...])
    x = _q(x_ref[...])
    s = _q(jax.lax.dot_general(a, u, (((0,), (0,)), ((), ())),
                               preferred_element_type=F32))
    r = _q(_dot(a, x))
    hp = _dot(jnp.concatenate([s, x], axis=1), w_ref[0, 0])
    hl = _dot(jnp.concatenate([r, u], axis=1), w_ref[0, 1])
    px_p = _q(_bn_relu(hp, g_ref[0, 0:1, :], be_ref[0, 0:1, :]))
    lx_p = _q(_bn_relu(hl, g_ref[0, 1:2, :], be_ref[0, 1:2, :]))

    sel = (jax.lax.broadcasted_iota(jnp.int32, (5, 1), 0) == t)
    w_p = _q(jnp.sum(jnp.where(sel, wp_ref[...], 0.0), axis=0, keepdims=True))
    w_l = _q(jnp.sum(jnp.where(sel, wl_ref[...], 0.0), axis=0, keepdims=True))
    ixg_ref[...] += w_p * lx_p
    lxg_ref[...] += w_l * px_p

    @pl.when(t == 4)
    def _():
        ixg_ref[...] += jnp.broadcast_to(bp_ref[...], ixg_ref.shape)
        lxg_ref[...] += jnp.broadcast_to(bl_ref[...], lxg_ref.shape)


def _k_head(ixg_ref, ixg2_ref, lxg_ref, lxg2_ref, pidx_ref, lidx_ref,
            wemb_ref, gemb_ref, beemb_ref, wout_ref, bout_ref, o_ref):
    pidx = pidx_ref[...]
    lidx = lidx_ref[...]
    ninf = jnp.float32(-jnp.inf)
    rows = []
    for s in range(16):
        pm = pidx == s
        lm = lidx == s
        r_ix = jnp.max(jnp.where(pm, ixg_ref[...], ninf), axis=0, keepdims=True)
        r_ix2 = jnp.max(jnp.where(pm, ixg2_ref[...], ninf), axis=0, keepdims=True)
        r_lx = jnp.max(jnp.where(lm, lxg_ref[...], ninf), axis=0, keepdims=True)
        r_lx2 = jnp.max(jnp.where(lm, lxg2_ref[...], ninf), axis=0, keepdims=True)
        rows.append(jnp.concatenate([r_ix, r_ix2, r_lx, r_lx2], axis=1))
    ilx = _q(jnp.concatenate(rows, axis=0))
    h = _dot(ilx, wemb_ref[...])
    mu = jnp.mean(h, axis=0, keepdims=True)
    var = jnp.mean((h - mu) ** 2, axis=0, keepdims=True)
    q = gemb_ref[...] * (h - mu) / jnp.sqrt(var + _EPS) + beemb_ref[...]
    q = jnp.where(q >= 0, q, 0.01 * q)
    o_ref[...] = _dot(q, wout_ref[...]) + bout_ref[...]


def _eirl_layer(plm, u, x, w, g, be, wp, bp, wl, bl, c):
    ixg, lxg = pl.pallas_call(
        functools.partial(_k_eirl, c=c),
        grid=(5,),
        in_specs=[
            pl.BlockSpec((1600, 768), lambda t: (0, 0)),
            pl.BlockSpec((1600, c), lambda t: (0, 0)),
            pl.BlockSpec((768, c), lambda t: (0, 0)),
            pl.BlockSpec((1, 2, 2 * c, 512), lambda t: (t, 0, 0, 0)),
            pl.BlockSpec((1, 2, 512), lambda t: (t, 0, 0)),
            pl.BlockSpec((1, 2, 512), lambda t: (t, 0, 0)),
            pl.BlockSpec((5, 1), lambda t: (0, 0)),
            pl.BlockSpec((1, 1), lambda t: (0, 0)),
            pl.BlockSpec((5, 1), lambda t: (0, 0)),
            pl.BlockSpec((1, 1), lambda t: (0, 0)),
        ],
        out_specs=[
            pl.BlockSpec((1600, 512), lambda t: (0, 0)),
            pl.BlockSpec((768, 512), lambda t: (0, 0)),
        ],
        out_shape=[
            jax.ShapeDtypeStruct((1600, 512), F32),
            jax.ShapeDtypeStruct((768, 512), F32),
        ],
        scratch_shapes=[pltpu.VMEM((1600, 768), F32),
                        pltpu.VMEM((1600, 768), F32)],
    )(plm, u, x, w, g, be, wp, bp, wl, bl)
    return ixg, lxg


def kernel(px, lx, pp_mat, pl_mat, ll_mat, res_idx, prot_g_idx, lig_g_idx,
           Win_p, bin_p, gin_p, bein_p, Win_l, bin_l, gin_l, bein_l,
           Wpg, bpg, gpg, bepg, Wlg, blg, glg, belg,
           e1_W, e1_b, e1_g, e1_be, e1_wp, e1_bp, e1_wl, e1_bl,
           e2_W, e2_b, e2_g, e2_be, e2_wp, e2_bp, e2_wl, e2_bl,
           Wemb, bemb, gemb, beemb, Wout, bout):
    row = lambda v: v.reshape(1, -1)
    col = lambda v: v.reshape(-1, 1)

    px1 = pl.pallas_call(
        _k_in_p,
        out_shape=jax.ShapeDtypeStruct((4800, 128), F32),
    )(px, Win_p, row(gin_p), row(bein_p))

    t = pl.pallas_call(
        _k_pp,
        grid=(10,),
        in_specs=[
            pl.BlockSpec((480, 4800), lambda i: (i, 0)),
            pl.BlockSpec((4800, 128), lambda i: (0, 0)),
        ],
        out_specs=pl.BlockSpec((480, 128), lambda i: (i, 0)),
        out_shape=jax.ShapeDtypeStruct((4800, 128), F32),
    )(pp_mat, px1)

    ix = pl.pallas_call(
        _k_gather,
        out_shape=jax.ShapeDtypeStruct((1600, 256), F32),
    )(t, Wpg, col(res_idx), row(gpg), row(bepg))

    lx2 = pl.pallas_call(
        _k_lig,
        out_shape=jax.ShapeDtypeStruct((768, 256), F32),
    )(lx, Win_l, row(gin_l), row(bein_l), ll_mat, Wlg, row(glg), row(belg))

    ixg, lxg = _eirl_layer(pl_mat, ix, lx2, e1_W, e1_g, e1_be,
                           col(e1_wp), col(e1_bp), col(e1_wl), col(e1_bl), 256)
    ixg2, lxg2 = _eirl_layer(pl_mat, ixg, lxg, e2_W, e2_g, e2_be,
                             col(e2_wp), col(e2_bp), col(e2_wl), col(e2_bl), 512)

    out = pl.pallas_call(
        _k_head,
        out_shape=jax.ShapeDtypeStruct((16, 1), F32),
    )(ixg, ixg2, lxg, lxg2, col(prot_g_idx), col(lig_g_idx),
      Wemb, row(gemb), row(beemb), Wout, row(bout))
    return out[:, 0]
